# baseline (speedup 1.0000x reference)
import functools

import jax
import jax.numpy as jnp
from jax.experimental import pallas as pl
from jax.experimental.pallas import tpu as pltpu

_SUBROWS = 32


def _argmax_sub(ref, r0, sr):
    c = ref.shape[1]
    val = ref[0, 0, pl.ds(r0, sr), :]
    idx = jnp.zeros(val.shape, dtype=jnp.int32)
    for k in range(1, c):
        cur = ref[0, k, pl.ds(r0, sr), :]
        gt = cur > val
        val = jnp.maximum(cur, val)
        idx = jnp.where(gt, jnp.int32(k), idx)
    return idx


def _iou_body(*refs, nsteps, cls_num):
    (*inrefs, score_ref, acc_ref) = refs
    step = pl.program_id(0) * pl.num_programs(1) + pl.program_id(1)

    @pl.when(step == 0)
    def _init():
        acc_ref[...] = jnp.zeros_like(acc_ref)

    tp = [jnp.int32(0)] * cls_num
    co = [jnp.int32(0)] * cls_num
    ct = [jnp.int32(0)] * cls_num
    subtiles = [(ref, r0) for ref in inrefs
                for r0 in range(0, ref.shape[2], _SUBROWS)]
    half = len(subtiles) // 2
    for (oref, r0), (tref, t0) in zip(subtiles[:half], subtiles[half:]):
        oi = _argmax_sub(oref, r0, _SUBROWS)
        ti = _argmax_sub(tref, t0, _SUBROWS)
        for c in range(cls_num):
            mo = oi == c
            mt = ti == c
            tp[c] = tp[c] + jnp.sum(mo & mt)
            co[c] = co[c] + jnp.sum(mo)
            ct[c] = ct[c] + jnp.sum(mt)

    rows = jax.lax.broadcasted_iota(jnp.int32, acc_ref.shape, 0)
    lanes = jax.lax.broadcasted_iota(jnp.int32, acc_ref.shape, 1)
    upd = jnp.zeros(acc_ref.shape, dtype=jnp.float32)
    for c in range(cls_num):
        at_c = lanes == c
        upd = upd + jnp.where((rows == 0) & at_c, tp[c].astype(jnp.float32), 0.0)
        upd = upd + jnp.where((rows == 1) & at_c, co[c].astype(jnp.float32), 0.0)
        upd = upd + jnp.where((rows == 2) & at_c, ct[c].astype(jnp.float32), 0.0)
    acc_ref[...] += upd

    @pl.when(step == nsteps - 1)
    def _finish():
        acc = acc_ref[...]
        tps = acc[0:1, :]
        denom = acc[1:2, :] + acc[2:3, :] - tps
        iou = jnp.where(denom > 0.0, tps / denom, 0.0)
        score_ref[...] = jnp.sum(iou, keepdims=True) / jnp.float32(cls_num)


def kernel(output, target):
    b, c, h, w = output.shape
    blk_r = 256
    n_r = h // blk_r
    nsteps = b * n_r

    body = functools.partial(_iou_body, nsteps=nsteps, cls_num=c)
    specs = [
        pl.BlockSpec((1, c, blk_r // 8, w),
                     functools.partial(lambda q, i, r: (i, 0, 8 * r + q, 0), q))
        for q in range(8)
    ]
    score = pl.pallas_call(
        body,
        grid=(b, n_r),
        in_specs=specs + specs,
        out_specs=pl.BlockSpec((1, 1), lambda i, r: (0, 0)),
        out_shape=jax.ShapeDtypeStruct((1, 1), jnp.float32),
        scratch_shapes=[pltpu.VMEM((3, 32), jnp.float32)],
    )(*([output] * 8 + [target] * 8))
    return score[0, 0]


# speedup vs baseline: 1.0033x; 1.0033x over previous
import functools

import jax
import jax.numpy as jnp
from jax.experimental import pallas as pl
from jax.experimental.pallas import tpu as pltpu

_SUBROWS = 32


def _argmax_sub(ref, r0, sr):
    c = ref.shape[1]
    val = ref[0, 0, pl.ds(r0, sr), :]
    idx = jnp.zeros(val.shape, dtype=jnp.int32)
    for k in range(1, c):
        cur = ref[0, k, pl.ds(r0, sr), :]
        gt = cur > val
        val = jnp.maximum(cur, val)
        idx = jnp.where(gt, jnp.int32(k), idx)
    return idx


def _iou_body(*refs, nsteps, cls_num):
    (o1, o2, o3, o4, t1, t2, t3, t4, score_ref, acc_ref) = refs
    step = pl.program_id(0) * pl.num_programs(1) + pl.program_id(1)

    @pl.when(step == 0)
    def _init():
        acc_ref[...] = jnp.zeros_like(acc_ref)

    tp = [jnp.int32(0)] * cls_num
    co = [jnp.int32(0)] * cls_num
    ct = [jnp.int32(0)] * cls_num
    subtiles = [(ref, r0) for ref in (o1, o2, o3, o4, t1, t2, t3, t4)
                for r0 in range(0, ref.shape[2], _SUBROWS)]
    half = len(subtiles) // 2
    for (oref, r0), (tref, t0) in zip(subtiles[:half], subtiles[half:]):
        oi = _argmax_sub(oref, r0, _SUBROWS)
        ti = _argmax_sub(tref, t0, _SUBROWS)
        for c in range(cls_num):
            mo = oi == c
            mt = ti == c
            tp[c] = tp[c] + jnp.sum(mo & mt)
            co[c] = co[c] + jnp.sum(mo)
            ct[c] = ct[c] + jnp.sum(mt)

    rows = jax.lax.broadcasted_iota(jnp.int32, acc_ref.shape, 0)
    lanes = jax.lax.broadcasted_iota(jnp.int32, acc_ref.shape, 1)
    upd = jnp.zeros(acc_ref.shape, dtype=jnp.float32)
    for c in range(cls_num):
        at_c = lanes == c
        upd = upd + jnp.where((rows == 0) & at_c, tp[c].astype(jnp.float32), 0.0)
        upd = upd + jnp.where((rows == 1) & at_c, co[c].astype(jnp.float32), 0.0)
        upd = upd + jnp.where((rows == 2) & at_c, ct[c].astype(jnp.float32), 0.0)
    acc_ref[...] += upd

    @pl.when(step == nsteps - 1)
    def _finish():
        acc = acc_ref[...]
        tps = acc[0:1, :]
        denom = acc[1:2, :] + acc[2:3, :] - tps
        iou = jnp.where(denom > 0.0, tps / denom, 0.0)
        score_ref[...] = jnp.sum(iou, keepdims=True) / jnp.float32(cls_num)


def kernel(output, target):
    b, c, h, w = output.shape
    blk_r = 256
    n_r = h // blk_r
    nsteps = b * n_r

    body = functools.partial(_iou_body, nsteps=nsteps, cls_num=c)
    specs = [
        pl.BlockSpec((1, c, blk_r // 4, w),
                     functools.partial(lambda q, i, r: (i, 0, 4 * r + q, 0), q))
        for q in range(4)
    ]
    score = pl.pallas_call(
        body,
        grid=(b, n_r),
        in_specs=specs + specs,
        out_specs=pl.BlockSpec((1, 1), lambda i, r: (0, 0)),
        out_shape=jax.ShapeDtypeStruct((1, 1), jnp.float32),
        scratch_shapes=[pltpu.VMEM((3, 32), jnp.float32)],
    )(output, output, output, output, target, target, target, target)
    return score[0, 0]
